# baseline (speedup 1.0000x reference)
import jax
import jax.numpy as jnp
from jax import lax
from jax.experimental import pallas as pl
from jax.experimental.pallas import tpu as pltpu

_NCH = 16
_CH = 1024
_NBUF = 4
_K = 3


def _tail(p4t, u):
    l0 = p4t[0:1, :]
    l1 = p4t[1:2, :]
    mu = p4t[2:3, :]
    s_raw = p4t[3:4, :]
    m = jnp.maximum(l0, l1)
    e0 = jnp.exp(l0 - m)
    e1 = jnp.exp(l1 - m)
    p_d = e0 / (e0 + e1)
    s = jax.nn.softplus(s_raw)
    p_rain = u[0:1, :]
    p_dist = u[1:2, :]
    ppf = jnp.exp(mu + s * (jnp.log(p_dist) - jnp.log1p(-p_dist)))
    return jnp.where(p_rain <= p_d, jnp.float32(0.0), ppf)


def _body(x_hbm, w1_ref, b1_ref, w2_ref, b2_ref, u_ref, out_ref, xbuf, sems):
    def copy(c):
        return pltpu.make_async_copy(
            x_hbm.at[pl.ds(c * _CH, _CH), :], xbuf.at[c % _NBUF], sems.at[c % _NBUF]
        )

    for c in range(_K):
        copy(c).start()
    w1 = w1_ref[...]
    w2 = w2_ref[...]
    b1 = b1_ref[...]
    b2c = b2_ref[...].reshape(4, 1)
    for c in range(_NCH):
        copy(c).wait()
        if c + _K < _NCH:
            copy(c + _K).start()
        h = jnp.tanh(
            jnp.dot(xbuf[c % _NBUF], w1, preferred_element_type=jnp.float32) + b1
        )
        p4t = lax.dot_general(
            w2, h, (((0,), (1,)), ((), ())),
            preferred_element_type=jnp.float32,
        ) + b2c
        u_c = u_ref[:, c * _CH:(c + 1) * _CH]
        out_ref[pl.ds(c * _CH, _CH)] = _tail(p4t, u_c).reshape(_CH)


def kernel(x, W1, b1, W2, b2, u):
    n, d_in = x.shape
    d_h = W1.shape[1]
    return pl.pallas_call(
        _body,
        in_specs=[
            pl.BlockSpec(memory_space=pl.ANY),
            pl.BlockSpec(memory_space=pltpu.VMEM),
            pl.BlockSpec(memory_space=pltpu.VMEM),
            pl.BlockSpec(memory_space=pltpu.VMEM),
            pl.BlockSpec(memory_space=pltpu.VMEM),
            pl.BlockSpec(memory_space=pltpu.VMEM),
        ],
        out_specs=pl.BlockSpec(memory_space=pltpu.VMEM),
        out_shape=jax.ShapeDtypeStruct((n,), jnp.float32),
        scratch_shapes=[
            pltpu.VMEM((_NBUF, _CH, d_in), jnp.float32),
            pltpu.SemaphoreType.DMA((_NBUF,)),
        ],
    )(x, W1, b1, W2, b2, u)


# speedup vs baseline: 1.1844x; 1.1844x over previous
import jax
import jax.numpy as jnp
from jax import lax
from jax.experimental import pallas as pl

_P = 4
_B = 512


def _tail(p4t, u):
    l0 = p4t[0:1, :]
    l1 = p4t[1:2, :]
    mu = p4t[2:3, :]
    s_raw = p4t[3:4, :]
    m = jnp.maximum(l0, l1)
    e0 = jnp.exp(l0 - m)
    e1 = jnp.exp(l1 - m)
    p_d = e0 / (e0 + e1)
    s = jax.nn.softplus(s_raw)
    p_rain = u[0:1, :]
    p_dist = u[1:2, :]
    ppf = jnp.exp(mu + s * (jnp.log(p_dist) - jnp.log1p(-p_dist)))
    return jnp.where(p_rain <= p_d, jnp.float32(0.0), ppf)


def _body(*refs):
    x_refs = refs[:_P]
    w1_ref, b1_ref, w2_ref, b2_ref, u_ref, out_ref = refs[_P:]
    w1 = w1_ref[...]
    w2 = w2_ref[...]
    b1 = b1_ref[...]
    b2c = b2_ref[...].reshape(4, 1)
    for p in range(_P):
        h = jnp.tanh(
            jnp.dot(x_refs[p][...], w1, preferred_element_type=jnp.float32) + b1
        )
        p4t = lax.dot_general(
            w2, h, (((0,), (1,)), ((), ())),
            preferred_element_type=jnp.float32,
        ) + b2c
        u_p = u_ref[:, p * _B:(p + 1) * _B]
        out_ref[pl.ds(p * _B, _B)] = _tail(p4t, u_p).reshape(_B)


def kernel(x, W1, b1, W2, b2, u):
    n, d_in = x.shape
    d_h = W1.shape[1]
    rows_per_step = _P * _B
    grid = (n // rows_per_step,)
    x_specs = [
        pl.BlockSpec((_B, d_in), lambda i, p=p: (i * _P + p, 0)) for p in range(_P)
    ]
    out = pl.pallas_call(
        _body,
        grid=grid,
        in_specs=x_specs + [
            pl.BlockSpec((d_in, d_h), lambda i: (0, 0)),
            pl.BlockSpec((d_h,), lambda i: (0,)),
            pl.BlockSpec((d_h, 4), lambda i: (0, 0)),
            pl.BlockSpec((4,), lambda i: (0,)),
            pl.BlockSpec((2, rows_per_step), lambda i: (0, i)),
        ],
        out_specs=pl.BlockSpec((rows_per_step,), lambda i: (i,)),
        out_shape=jax.ShapeDtypeStruct((n,), jnp.float32),
    )(*([x] * _P), W1, b1, W2, b2, u)
    return out


# speedup vs baseline: 1.3882x; 1.1720x over previous
import jax
import jax.numpy as jnp
from jax import lax
from jax.experimental import pallas as pl

_P = 2
_B = 2048


def _tail(p4t, u):
    l0 = p4t[0:1, :]
    l1 = p4t[1:2, :]
    mu = p4t[2:3, :]
    s_raw = p4t[3:4, :]
    m = jnp.maximum(l0, l1)
    e0 = jnp.exp(l0 - m)
    e1 = jnp.exp(l1 - m)
    p_d = e0 / (e0 + e1)
    s = jax.nn.softplus(s_raw)
    p_rain = u[0:1, :]
    p_dist = u[1:2, :]
    ppf = jnp.exp(mu + s * (jnp.log(p_dist) - jnp.log1p(-p_dist)))
    return jnp.where(p_rain <= p_d, jnp.float32(0.0), ppf)


def _body(*refs):
    x_refs = refs[:_P]
    w1_ref, b1_ref, w2_ref, b2_ref, u_ref, out_ref = refs[_P:]
    w1 = w1_ref[...]
    w2 = w2_ref[...]
    b1 = b1_ref[...]
    b2c = b2_ref[...].reshape(4, 1)
    for p in range(_P):
        h = jnp.tanh(
            jnp.dot(x_refs[p][...], w1, preferred_element_type=jnp.float32) + b1
        )
        p4t = lax.dot_general(
            w2, h, (((0,), (1,)), ((), ())),
            preferred_element_type=jnp.float32,
        ) + b2c
        u_p = u_ref[:, p * _B:(p + 1) * _B]
        out_ref[pl.ds(p * _B, _B)] = _tail(p4t, u_p).reshape(_B)


def kernel(x, W1, b1, W2, b2, u):
    n, d_in = x.shape
    d_h = W1.shape[1]
    rows_per_step = _P * _B
    grid = (n // rows_per_step,)
    x_specs = [
        pl.BlockSpec((_B, d_in), lambda i, p=p: (i * _P + p, 0)) for p in range(_P)
    ]
    out = pl.pallas_call(
        _body,
        grid=grid,
        in_specs=x_specs + [
            pl.BlockSpec((d_in, d_h), lambda i: (0, 0)),
            pl.BlockSpec((d_h,), lambda i: (0,)),
            pl.BlockSpec((d_h, 4), lambda i: (0, 0)),
            pl.BlockSpec((4,), lambda i: (0,)),
            pl.BlockSpec((2, rows_per_step), lambda i: (0, i)),
        ],
        out_specs=pl.BlockSpec((rows_per_step,), lambda i: (i,)),
        out_shape=jax.ShapeDtypeStruct((n,), jnp.float32),
    )(*([x] * _P), W1, b1, W2, b2, u)
    return out
